# baseline (speedup 1.0000x reference)
import jax
import jax.numpy as jnp
from jax.experimental import pallas as pl
from jax.experimental.pallas import tpu as pltpu

N_U = 4096
N_I = 4096
DD = 128
E_EDGES = 131072


def _fused_side_body(q_ref, k_ref, v_ref, watt_ref, m2a_ref, m2b_ref,
                     wmean_ref, oa_ref, ob_ref, wobs_ref, out_ref):
    w = watt_ref[...]
    m = jnp.dot(w, w.T, preferred_element_type=jnp.float32)
    q = jnp.dot(q_ref[...], m, preferred_element_type=jnp.float32)
    logits = jax.lax.dot_general(q, k_ref[...], (((1,), (1,)), ((), ())),
                                 preferred_element_type=jnp.float32)
    mx = jnp.max(logits, axis=1, keepdims=True)
    p = jnp.exp(logits - mx)
    s = jnp.sum(p, axis=1, keepdims=True)
    att = jnp.dot(p, v_ref[...], preferred_element_type=jnp.float32) / s
    h1 = jnp.dot(att, w, preferred_element_type=jnp.float32)
    h2 = jnp.dot(m2a_ref[...] * m2b_ref[...], wmean_ref[...],
                 preferred_element_type=jnp.float32)
    ho = jnp.tanh(jnp.dot(oa_ref[...] * ob_ref[...], wobs_ref[...],
                          preferred_element_type=jnp.float32))
    h = jnp.tanh(jnp.concatenate([h1, h2, ho], axis=1))
    n = jnp.sqrt(jnp.sum(h * h, axis=1, keepdims=True))
    out_ref[...] = h / jnp.maximum(n, 1e-12)


def _fused_side(q, k, v, watt, m2a, m2b, wmean, oa, ob, wobs):
    bsz = q.shape[0]
    bq = 256
    qmap = lambda i: (i, 0)
    full = lambda i: (0, 0)
    return pl.pallas_call(
        _fused_side_body,
        grid=(bsz // bq,),
        in_specs=[
            pl.BlockSpec((bq, DD), qmap),
            pl.BlockSpec((N_U, DD), full),
            pl.BlockSpec((N_U, DD), full),
            pl.BlockSpec((DD, DD), full),
            pl.BlockSpec((bq, DD), qmap),
            pl.BlockSpec((bq, DD), qmap),
            pl.BlockSpec((DD, DD), full),
            pl.BlockSpec((bq, DD), qmap),
            pl.BlockSpec((bq, DD), qmap),
            pl.BlockSpec((DD, DD), full),
        ],
        out_specs=pl.BlockSpec((bq, 3 * DD), qmap),
        out_shape=jax.ShapeDtypeStruct((bsz, 3 * DD), jnp.float32),
    )(q, k, v, watt, m2a, m2b, wmean, oa, ob, wobs)


def _segsum(vals, x_rows, seg_ids, n_out):
    return jax.ops.segment_sum(vals[:, None] * x_rows, seg_ids,
                               num_segments=n_out)


def kernel(user_emb, item_emb, W_att_user, W_att_item, W_mean_user,
           W_mean_item, W_obs_user, W_obs_item, sample_user_n_j,
           sample_item_n_j, obs_user_n_j, obs_item_n_j, adj_values,
           obs_adj_values, users, pos_items, neg_items, obs_users,
           obs_pos_items, obs_neg_items, adj_indices, obs_adj_indices):
    spmm_u = _segsum(adj_values, item_emb[adj_indices[1]], adj_indices[0], N_U)
    spmm_ti = _segsum(adj_values, user_emb[adj_indices[0]], adj_indices[1], N_I)
    spmm_ou = _segsum(obs_adj_values, item_emb[obs_adj_indices[1]],
                      obs_adj_indices[0], N_U)
    spmm_oi = _segsum(obs_adj_values, user_emb[obs_adj_indices[0]],
                      obs_adj_indices[1], N_I)

    idx2 = jnp.concatenate([pos_items, neg_items])
    idxo2 = jnp.concatenate([obs_pos_items, obs_neg_items])

    h_u = _fused_side(user_emb[users], spmm_ti, item_emb, W_att_user,
                      spmm_u[users], sample_user_n_j[users], W_mean_user,
                      spmm_ou[obs_users], obs_user_n_j[obs_users], W_obs_user)
    h_pn = _fused_side(item_emb[idx2], spmm_u, user_emb, W_att_item,
                       spmm_ti[idx2], sample_item_n_j[idx2], W_mean_item,
                       spmm_oi[idxo2], obs_item_n_j[idxo2], W_obs_item)
    return h_u, h_pn[:1024], h_pn[1024:]


# speedup vs baseline: 4.3884x; 4.3884x over previous
import functools

import jax
import jax.numpy as jnp
from jax import lax
from jax.experimental import pallas as pl
from jax.experimental.pallas import tpu as pltpu
from jax.experimental.pallas import tpu_sc as plsc

N_U = 4096
N_I = 4096
DD = 128
E_EDGES = 131072

_SC_NS = 16
_EPT = E_EDGES // _SC_NS
_CK = 128


def _sc_spmm(tbl_stack, gidx_stack, sidx_stack, vals_stack):
    mesh = plsc.VectorSubcoreMesh(core_axis_name="c", subcore_axis_name="s")

    @functools.partial(
        pl.kernel,
        out_type=jax.ShapeDtypeStruct((2, 2, N_U, DD), jnp.float32),
        mesh=mesh,
        compiler_params=pltpu.CompilerParams(needs_layout_passes=False),
        scratch_types=[
            pltpu.VMEM_SHARED((N_U, DD), jnp.float32),
            pltpu.VMEM_SHARED((N_U, DD), jnp.float32),
            pltpu.VMEM((_CK, DD), jnp.float32),
            pltpu.VMEM((_CK,), jnp.int32),
            pltpu.VMEM((_CK,), jnp.int32),
            pltpu.VMEM((_CK,), jnp.float32),
            pltpu.VMEM((64, DD), jnp.float32),
        ],
    )
    def k(tbl_hbm, gidx_hbm, sidx_hbm, vals_hbm, out_hbm,
          acc0, acc1, rows_v, gidx_v, sidx_v, vals_v, zbuf):
        c = lax.axis_index("c")
        s = lax.axis_index("s")

        @pl.loop(0, 64)
        def _(i):
            for j in range(8):
                zbuf[i, pl.ds(16 * j, 16)] = jnp.zeros((16,), jnp.float32)

        for acc in (acc0, acc1):
            for r in range(4):
                pltpu.sync_copy(zbuf, acc.at[pl.ds(s * 256 + r * 64, 64)])
        plsc.subcore_barrier()

        for t, acc in ((0, acc0), (1, acc1)):
            @pl.loop(0, _EPT, step=_CK)
            def _(off, t=t, acc=acc):
                base = s * _EPT + off
                pltpu.sync_copy(gidx_hbm.at[c, t, pl.ds(base, _CK)], gidx_v)
                pltpu.sync_copy(sidx_hbm.at[c, t, pl.ds(base, _CK)], sidx_v)
                pltpu.sync_copy(vals_hbm.at[t, pl.ds(base, _CK)], vals_v)
                pltpu.sync_copy(tbl_hbm.at[c].at[gidx_v], rows_v)

                @pl.loop(0, _CK)
                def _(i):
                    iv = jnp.broadcast_to(i.astype(jnp.int32), (16,))
                    v = plsc.load_gather(vals_v, [iv])
                    for j in range(8):
                        sl = pl.ds(16 * j, 16)
                        rows_v[i, sl] = rows_v[i, sl] * v

                pltpu.sync_copy(rows_v, acc.at[sidx_v], add=True)
        plsc.subcore_barrier()

        for t, acc in ((0, acc0), (1, acc1)):
            pltpu.sync_copy(acc.at[pl.ds(s * 256, 256)],
                            out_hbm.at[c, t, pl.ds(s * 256, 256)])

    return k(tbl_stack, gidx_stack, sidx_stack, vals_stack)


def _fused_side_body(q_ref, k_ref, v_ref, watt_ref, m2a_ref, m2b_ref,
                     wmean_ref, oa_ref, ob_ref, wobs_ref, out_ref):
    w = watt_ref[...]
    m = jnp.dot(w, w.T, preferred_element_type=jnp.float32)
    q = jnp.dot(q_ref[...], m, preferred_element_type=jnp.float32)
    logits = jax.lax.dot_general(q, k_ref[...], (((1,), (1,)), ((), ())),
                                 preferred_element_type=jnp.float32)
    mx = jnp.max(logits, axis=1, keepdims=True)
    p = jnp.exp(logits - mx)
    s = jnp.sum(p, axis=1, keepdims=True)
    att = jnp.dot(p, v_ref[...], preferred_element_type=jnp.float32) / s
    h1 = jnp.dot(att, w, preferred_element_type=jnp.float32)
    h2 = jnp.dot(m2a_ref[...] * m2b_ref[...], wmean_ref[...],
                 preferred_element_type=jnp.float32)
    ho = jnp.tanh(jnp.dot(oa_ref[...] * ob_ref[...], wobs_ref[...],
                          preferred_element_type=jnp.float32))
    h = jnp.tanh(jnp.concatenate([h1, h2, ho], axis=1))
    n = jnp.sqrt(jnp.sum(h * h, axis=1, keepdims=True))
    out_ref[...] = h / jnp.maximum(n, 1e-12)


def _fused_side(q, k, v, watt, m2a, m2b, wmean, oa, ob, wobs):
    bsz = q.shape[0]
    bq = 256
    qmap = lambda i: (i, 0)
    full = lambda i: (0, 0)
    return pl.pallas_call(
        _fused_side_body,
        grid=(bsz // bq,),
        in_specs=[
            pl.BlockSpec((bq, DD), qmap),
            pl.BlockSpec((N_U, DD), full),
            pl.BlockSpec((N_U, DD), full),
            pl.BlockSpec((DD, DD), full),
            pl.BlockSpec((bq, DD), qmap),
            pl.BlockSpec((bq, DD), qmap),
            pl.BlockSpec((DD, DD), full),
            pl.BlockSpec((bq, DD), qmap),
            pl.BlockSpec((bq, DD), qmap),
            pl.BlockSpec((DD, DD), full),
        ],
        out_specs=pl.BlockSpec((bq, 3 * DD), qmap),
        out_shape=jax.ShapeDtypeStruct((bsz, 3 * DD), jnp.float32),
    )(q, k, v, watt, m2a, m2b, wmean, oa, ob, wobs)


def kernel(user_emb, item_emb, W_att_user, W_att_item, W_mean_user,
           W_mean_item, W_obs_user, W_obs_item, sample_user_n_j,
           sample_item_n_j, obs_user_n_j, obs_item_n_j, adj_values,
           obs_adj_values, users, pos_items, neg_items, obs_users,
           obs_pos_items, obs_neg_items, adj_indices, obs_adj_indices):
    tbl_stack = jnp.stack([item_emb, user_emb])
    gidx_stack = jnp.stack([
        jnp.stack([adj_indices[1], obs_adj_indices[1]]),
        jnp.stack([adj_indices[0], obs_adj_indices[0]]),
    ])
    sidx_stack = jnp.stack([
        jnp.stack([adj_indices[0], obs_adj_indices[0]]),
        jnp.stack([adj_indices[1], obs_adj_indices[1]]),
    ])
    vals_stack = jnp.stack([adj_values, obs_adj_values])
    spmm_o = _sc_spmm(tbl_stack, gidx_stack, sidx_stack, vals_stack)
    spmm_u, spmm_ou = spmm_o[0, 0], spmm_o[0, 1]
    spmm_ti, spmm_oi = spmm_o[1, 0], spmm_o[1, 1]

    idx2 = jnp.concatenate([pos_items, neg_items])
    idxo2 = jnp.concatenate([obs_pos_items, obs_neg_items])

    h_u = _fused_side(user_emb[users], spmm_ti, item_emb, W_att_user,
                      spmm_u[users], sample_user_n_j[users], W_mean_user,
                      spmm_ou[obs_users], obs_user_n_j[obs_users], W_obs_user)
    h_pn = _fused_side(item_emb[idx2], spmm_u, user_emb, W_att_item,
                       spmm_ti[idx2], sample_item_n_j[idx2], W_mean_item,
                       spmm_oi[idxo2], obs_item_n_j[idxo2], W_obs_item)
    return h_u, h_pn[:1024], h_pn[1024:]


# speedup vs baseline: 8.1459x; 1.8562x over previous
import functools

import jax
import jax.numpy as jnp
from jax import lax
from jax.experimental import pallas as pl
from jax.experimental.pallas import tpu as pltpu
from jax.experimental.pallas import tpu_sc as plsc

N_U = 4096
N_I = 4096
DD = 128
E_EDGES = 131072

_SC_NS = 16
_EPT = E_EDGES // _SC_NS
_CK = 128


def _sc_spmm(tbl_stack, gidx_stack, sidx_stack, vals_stack):
    mesh = plsc.VectorSubcoreMesh(core_axis_name="c", subcore_axis_name="s")

    n_chunks = _EPT // _CK

    @functools.partial(
        pl.kernel,
        out_type=jax.ShapeDtypeStruct((2, 2, N_U, DD), jnp.float32),
        mesh=mesh,
        compiler_params=pltpu.CompilerParams(needs_layout_passes=False),
        scratch_types=[
            pltpu.VMEM_SHARED((N_U, DD), jnp.float32),
            pltpu.VMEM_SHARED((N_U, DD), jnp.float32),
            pltpu.VMEM((_CK, DD), jnp.float32),
            pltpu.VMEM((_CK, DD), jnp.float32),
            pltpu.VMEM((n_chunks, _CK), jnp.int32),
            pltpu.VMEM((n_chunks, _CK), jnp.int32),
            pltpu.VMEM((_EPT,), jnp.float32),
            pltpu.VMEM((64, DD), jnp.float32),
            pltpu.SemaphoreType.DMA,
            pltpu.SemaphoreType.DMA,
        ],
    )
    def k(tbl_hbm, gidx_hbm, sidx_hbm, vals_hbm, out_hbm,
          acc0, acc1, rows0, rows1, gidx_all, sidx_all, vals_f, zbuf,
          sem0, sem1):
        c = lax.axis_index("c")
        s = lax.axis_index("s")
        tbl_c = tbl_hbm.at[c]

        @pl.loop(0, 64)
        def _(i):
            for j in range(8):
                zbuf[i, pl.ds(16 * j, 16)] = jnp.zeros((16,), jnp.float32)

        for acc in (acc0, acc1):
            for r in range(4):
                pltpu.sync_copy(zbuf, acc.at[pl.ds(s * 256 + r * 64, 64)])
        plsc.subcore_barrier()

        def mul_chunk(jc, rows_b):
            @pl.loop(0, _CK)
            def _(i):
                iv = jnp.broadcast_to((jc * _CK + i).astype(jnp.int32), (16,))
                v = plsc.load_gather(vals_f, [iv])
                for j in range(8):
                    sl = pl.ds(16 * j, 16)
                    rows_b[i, sl] = rows_b[i, sl] * v

        for t, acc in ((0, acc0), (1, acc1)):
            pltpu.sync_copy(gidx_hbm.at[c, t, s], gidx_all)
            pltpu.sync_copy(sidx_hbm.at[c, t, s], sidx_all)
            pltpu.sync_copy(vals_hbm.at[t, pl.ds(s * _EPT, _EPT)], vals_f)
            pltpu.async_copy(tbl_c.at[gidx_all.at[0]], rows0, sem0)

            def chunk(jc, rows_b, sem_b, acc=acc):
                pltpu.make_async_copy(tbl_c.at[gidx_all.at[jc]], rows_b,
                                      sem_b).wait()
                mul_chunk(jc, rows_b)
                pltpu.sync_copy(rows_b, acc.at[sidx_all.at[jc]], add=True)

            @pl.loop(0, n_chunks, step=2)
            def _(j, acc=acc):
                pltpu.async_copy(tbl_c.at[gidx_all.at[j + 1]], rows1, sem1)
                chunk(j, rows0, sem0)

                @pl.when(j + 2 < n_chunks)
                def _():
                    pltpu.async_copy(tbl_c.at[gidx_all.at[j + 2]], rows0, sem0)
                chunk(j + 1, rows1, sem1)
        plsc.subcore_barrier()

        for t, acc in ((0, acc0), (1, acc1)):
            pltpu.sync_copy(acc.at[pl.ds(s * 256, 256)],
                            out_hbm.at[c, t, pl.ds(s * 256, 256)])

    return k(tbl_stack, gidx_stack, sidx_stack, vals_stack)


def _fused_side_body(q_ref, k_ref, v_ref, watt_ref, m2a_ref, m2b_ref,
                     wmean_ref, oa_ref, ob_ref, wobs_ref, out_ref):
    w = watt_ref[...]
    m = jnp.dot(w, w.T, preferred_element_type=jnp.float32)
    q = jnp.dot(q_ref[...], m, preferred_element_type=jnp.float32)
    logits = jax.lax.dot_general(q, k_ref[...], (((1,), (1,)), ((), ())),
                                 preferred_element_type=jnp.float32)
    mx = jnp.max(logits, axis=1, keepdims=True)
    p = jnp.exp(logits - mx)
    s = jnp.sum(p, axis=1, keepdims=True)
    att = jnp.dot(p, v_ref[...], preferred_element_type=jnp.float32) / s
    h1 = jnp.dot(att, w, preferred_element_type=jnp.float32)
    h2 = jnp.dot(m2a_ref[...] * m2b_ref[...], wmean_ref[...],
                 preferred_element_type=jnp.float32)
    ho = jnp.tanh(jnp.dot(oa_ref[...] * ob_ref[...], wobs_ref[...],
                          preferred_element_type=jnp.float32))
    h = jnp.tanh(jnp.concatenate([h1, h2, ho], axis=1))
    n = jnp.sqrt(jnp.sum(h * h, axis=1, keepdims=True))
    out_ref[...] = h / jnp.maximum(n, 1e-12)


def _fused_side(q, k, v, watt, m2a, m2b, wmean, oa, ob, wobs):
    bsz = q.shape[0]
    bq = 256
    qmap = lambda i: (i, 0)
    full = lambda i: (0, 0)
    return pl.pallas_call(
        _fused_side_body,
        grid=(bsz // bq,),
        in_specs=[
            pl.BlockSpec((bq, DD), qmap),
            pl.BlockSpec((N_U, DD), full),
            pl.BlockSpec((N_U, DD), full),
            pl.BlockSpec((DD, DD), full),
            pl.BlockSpec((bq, DD), qmap),
            pl.BlockSpec((bq, DD), qmap),
            pl.BlockSpec((DD, DD), full),
            pl.BlockSpec((bq, DD), qmap),
            pl.BlockSpec((bq, DD), qmap),
            pl.BlockSpec((DD, DD), full),
        ],
        out_specs=pl.BlockSpec((bq, 3 * DD), qmap),
        out_shape=jax.ShapeDtypeStruct((bsz, 3 * DD), jnp.float32),
    )(q, k, v, watt, m2a, m2b, wmean, oa, ob, wobs)


def kernel(user_emb, item_emb, W_att_user, W_att_item, W_mean_user,
           W_mean_item, W_obs_user, W_obs_item, sample_user_n_j,
           sample_item_n_j, obs_user_n_j, obs_item_n_j, adj_values,
           obs_adj_values, users, pos_items, neg_items, obs_users,
           obs_pos_items, obs_neg_items, adj_indices, obs_adj_indices):
    tbl_stack = jnp.stack([item_emb, user_emb])
    n_chunks = _EPT // _CK
    gidx_stack = jnp.stack([
        jnp.stack([adj_indices[1], obs_adj_indices[1]]),
        jnp.stack([adj_indices[0], obs_adj_indices[0]]),
    ]).reshape(2, 2, _SC_NS, n_chunks, _CK)
    sidx_stack = jnp.stack([
        jnp.stack([adj_indices[0], obs_adj_indices[0]]),
        jnp.stack([adj_indices[1], obs_adj_indices[1]]),
    ]).reshape(2, 2, _SC_NS, n_chunks, _CK)
    vals_stack = jnp.stack([adj_values, obs_adj_values])
    spmm_o = _sc_spmm(tbl_stack, gidx_stack, sidx_stack, vals_stack)
    spmm_u, spmm_ou = spmm_o[0, 0], spmm_o[0, 1]
    spmm_ti, spmm_oi = spmm_o[1, 0], spmm_o[1, 1]

    idx2 = jnp.concatenate([pos_items, neg_items])
    idxo2 = jnp.concatenate([obs_pos_items, obs_neg_items])

    h_u = _fused_side(user_emb[users], spmm_ti, item_emb, W_att_user,
                      spmm_u[users], sample_user_n_j[users], W_mean_user,
                      spmm_ou[obs_users], obs_user_n_j[obs_users], W_obs_user)
    h_pn = _fused_side(item_emb[idx2], spmm_u, user_emb, W_att_item,
                       spmm_ti[idx2], sample_item_n_j[idx2], W_mean_item,
                       spmm_oi[idxo2], obs_item_n_j[idxo2], W_obs_item)
    return h_u, h_pn[:1024], h_pn[1024:]


# speedup vs baseline: 9.8736x; 1.2121x over previous
import functools

import jax
import jax.numpy as jnp
from jax import lax
from jax.experimental import pallas as pl
from jax.experimental.pallas import tpu as pltpu
from jax.experimental.pallas import tpu_sc as plsc

N_U = 4096
N_I = 4096
DD = 128
E_EDGES = 131072

_SC_NS = 16
_EPT = E_EDGES // _SC_NS
_CK = 128


def _sc_spmm(tbl_stack, gidx_stack, sidx_stack, vals_stack):
    mesh = plsc.VectorSubcoreMesh(core_axis_name="c", subcore_axis_name="s")

    n_chunks = _EPT // _CK

    @functools.partial(
        pl.kernel,
        out_type=jax.ShapeDtypeStruct((2, 2, N_U, DD), jnp.float32),
        mesh=mesh,
        compiler_params=pltpu.CompilerParams(needs_layout_passes=False),
        scratch_types=[
            pltpu.VMEM_SHARED((N_U, DD), jnp.float32),
            pltpu.VMEM_SHARED((N_U, DD), jnp.float32),
            pltpu.VMEM((_CK, DD), jnp.float32),
            pltpu.VMEM((_CK, DD), jnp.float32),
            pltpu.VMEM((n_chunks, _CK), jnp.int32),
            pltpu.VMEM((n_chunks, _CK), jnp.int32),
            pltpu.VMEM((_EPT,), jnp.float32),
            pltpu.VMEM((64, DD), jnp.float32),
            pltpu.SemaphoreType.DMA,
            pltpu.SemaphoreType.DMA,
        ],
    )
    def k(tbl_hbm, gidx_hbm, sidx_hbm, vals_hbm, out_hbm,
          acc0, acc1, rows0, rows1, gidx_all, sidx_all, vals_f, zbuf,
          sem0, sem1):
        c = lax.axis_index("c")
        s = lax.axis_index("s")
        tbl_c = tbl_hbm.at[c]

        @pl.loop(0, 64)
        def _(i):
            for j in range(8):
                zbuf[i, pl.ds(16 * j, 16)] = jnp.zeros((16,), jnp.float32)

        for acc in (acc0, acc1):
            for r in range(4):
                pltpu.sync_copy(zbuf, acc.at[pl.ds(s * 256 + r * 64, 64)])
        plsc.subcore_barrier()

        dnums = lax.GatherDimensionNumbers(
            offset_dims=(), collapsed_slice_dims=(0,), start_index_map=(0,))

        def mul_chunk(jc, rows_b):
            @pl.loop(0, _CK // 16)
            def _(g):
                v16 = vals_f[pl.ds(jc * _CK + g * 16, 16)]
                for l in range(16):
                    vb = lax.gather(
                        v16, jnp.full((16, 1), l, jnp.int32), dnums,
                        slice_sizes=(1,),
                        mode=lax.GatherScatterMode.PROMISE_IN_BOUNDS)
                    i = g * 16 + l
                    for j in range(8):
                        sl = pl.ds(16 * j, 16)
                        rows_b[i, sl] = rows_b[i, sl] * vb

        for t, acc in ((0, acc0), (1, acc1)):
            pltpu.sync_copy(gidx_hbm.at[c, t, s], gidx_all)
            pltpu.sync_copy(sidx_hbm.at[c, t, s], sidx_all)
            pltpu.sync_copy(vals_hbm.at[t, pl.ds(s * _EPT, _EPT)], vals_f)
            pltpu.async_copy(tbl_c.at[gidx_all.at[0]], rows0, sem0)

            def chunk(jc, rows_b, sem_b, acc=acc):
                pltpu.make_async_copy(tbl_c.at[gidx_all.at[jc]], rows_b,
                                      sem_b).wait()
                mul_chunk(jc, rows_b)
                pltpu.sync_copy(rows_b, acc.at[sidx_all.at[jc]], add=True)

            @pl.loop(0, n_chunks, step=2)
            def _(j, acc=acc):
                pltpu.async_copy(tbl_c.at[gidx_all.at[j + 1]], rows1, sem1)
                chunk(j, rows0, sem0)

                @pl.when(j + 2 < n_chunks)
                def _():
                    pltpu.async_copy(tbl_c.at[gidx_all.at[j + 2]], rows0, sem0)
                chunk(j + 1, rows1, sem1)
        plsc.subcore_barrier()

        for t, acc in ((0, acc0), (1, acc1)):
            pltpu.sync_copy(acc.at[pl.ds(s * 256, 256)],
                            out_hbm.at[c, t, pl.ds(s * 256, 256)])

    return k(tbl_stack, gidx_stack, sidx_stack, vals_stack)


def _fused_side_body(q_ref, k_ref, v_ref, watt_ref, m2a_ref, m2b_ref,
                     wmean_ref, oa_ref, ob_ref, wobs_ref, out_ref):
    w = watt_ref[...]
    m = jnp.dot(w, w.T, preferred_element_type=jnp.float32)
    q = jnp.dot(q_ref[...], m, preferred_element_type=jnp.float32)
    logits = jax.lax.dot_general(q, k_ref[...], (((1,), (1,)), ((), ())),
                                 preferred_element_type=jnp.float32)
    mx = jnp.max(logits, axis=1, keepdims=True)
    p = jnp.exp(logits - mx)
    s = jnp.sum(p, axis=1, keepdims=True)
    att = jnp.dot(p, v_ref[...], preferred_element_type=jnp.float32) / s
    h1 = jnp.dot(att, w, preferred_element_type=jnp.float32)
    h2 = jnp.dot(m2a_ref[...] * m2b_ref[...], wmean_ref[...],
                 preferred_element_type=jnp.float32)
    ho = jnp.tanh(jnp.dot(oa_ref[...] * ob_ref[...], wobs_ref[...],
                          preferred_element_type=jnp.float32))
    h = jnp.tanh(jnp.concatenate([h1, h2, ho], axis=1))
    n = jnp.sqrt(jnp.sum(h * h, axis=1, keepdims=True))
    out_ref[...] = h / jnp.maximum(n, 1e-12)


def _fused_side(q, k, v, watt, m2a, m2b, wmean, oa, ob, wobs):
    bsz = q.shape[0]
    bq = 256
    qmap = lambda i: (i, 0)
    full = lambda i: (0, 0)
    return pl.pallas_call(
        _fused_side_body,
        grid=(bsz // bq,),
        in_specs=[
            pl.BlockSpec((bq, DD), qmap),
            pl.BlockSpec((N_U, DD), full),
            pl.BlockSpec((N_U, DD), full),
            pl.BlockSpec((DD, DD), full),
            pl.BlockSpec((bq, DD), qmap),
            pl.BlockSpec((bq, DD), qmap),
            pl.BlockSpec((DD, DD), full),
            pl.BlockSpec((bq, DD), qmap),
            pl.BlockSpec((bq, DD), qmap),
            pl.BlockSpec((DD, DD), full),
        ],
        out_specs=pl.BlockSpec((bq, 3 * DD), qmap),
        out_shape=jax.ShapeDtypeStruct((bsz, 3 * DD), jnp.float32),
    )(q, k, v, watt, m2a, m2b, wmean, oa, ob, wobs)


def kernel(user_emb, item_emb, W_att_user, W_att_item, W_mean_user,
           W_mean_item, W_obs_user, W_obs_item, sample_user_n_j,
           sample_item_n_j, obs_user_n_j, obs_item_n_j, adj_values,
           obs_adj_values, users, pos_items, neg_items, obs_users,
           obs_pos_items, obs_neg_items, adj_indices, obs_adj_indices):
    tbl_stack = jnp.stack([item_emb, user_emb])
    n_chunks = _EPT // _CK
    gidx_stack = jnp.stack([
        jnp.stack([adj_indices[1], obs_adj_indices[1]]),
        jnp.stack([adj_indices[0], obs_adj_indices[0]]),
    ]).reshape(2, 2, _SC_NS, n_chunks, _CK)
    sidx_stack = jnp.stack([
        jnp.stack([adj_indices[0], obs_adj_indices[0]]),
        jnp.stack([adj_indices[1], obs_adj_indices[1]]),
    ]).reshape(2, 2, _SC_NS, n_chunks, _CK)
    vals_stack = jnp.stack([adj_values, obs_adj_values])
    spmm_o = _sc_spmm(tbl_stack, gidx_stack, sidx_stack, vals_stack)
    spmm_u, spmm_ou = spmm_o[0, 0], spmm_o[0, 1]
    spmm_ti, spmm_oi = spmm_o[1, 0], spmm_o[1, 1]

    idx2 = jnp.concatenate([pos_items, neg_items])
    idxo2 = jnp.concatenate([obs_pos_items, obs_neg_items])

    h_u = _fused_side(user_emb[users], spmm_ti, item_emb, W_att_user,
                      spmm_u[users], sample_user_n_j[users], W_mean_user,
                      spmm_ou[obs_users], obs_user_n_j[obs_users], W_obs_user)
    h_pn = _fused_side(item_emb[idx2], spmm_u, user_emb, W_att_item,
                       spmm_ti[idx2], sample_item_n_j[idx2], W_mean_item,
                       spmm_oi[idxo2], obs_item_n_j[idxo2], W_obs_item)
    return h_u, h_pn[:1024], h_pn[1024:]


# speedup vs baseline: 10.4583x; 1.0592x over previous
import functools

import jax
import jax.numpy as jnp
from jax import lax
from jax.experimental import pallas as pl
from jax.experimental.pallas import tpu as pltpu
from jax.experimental.pallas import tpu_sc as plsc

N_U = 4096
N_I = 4096
DD = 128
E_EDGES = 131072

_SC_NS = 16
_EPT = E_EDGES // _SC_NS
_CK = 128


def _sc_spmm(tbl_stack, gidx_stack, sidx_stack, vals_stack,
             sample_user_n_j, obs_user_n_j, sample_item_n_j, obs_item_n_j,
             i_users, i_obsu, i_idx2, i_idxo2):
    mesh = plsc.VectorSubcoreMesh(core_axis_name="c", subcore_axis_name="s")

    n_chunks = _EPT // _CK

    f32 = jnp.float32
    gout = lambda n: jax.ShapeDtypeStruct((n, DD), f32)

    @functools.partial(
        pl.kernel,
        out_type=[jax.ShapeDtypeStruct((2, 2, N_U, DD), f32),
                  gout(1024), gout(1024), gout(1024), gout(1024), gout(1024),
                  gout(2048), gout(2048), gout(2048), gout(2048), gout(2048)],
        mesh=mesh,
        compiler_params=pltpu.CompilerParams(needs_layout_passes=False),
        scratch_types=[
            pltpu.VMEM_SHARED((N_U, DD), jnp.float32),
            pltpu.VMEM((_CK, DD), jnp.float32),
            pltpu.VMEM((_CK, DD), jnp.float32),
            pltpu.VMEM((n_chunks, _CK), jnp.int32),
            pltpu.VMEM((n_chunks, _CK), jnp.int32),
            pltpu.VMEM((_EPT,), jnp.float32),
            pltpu.VMEM((64, DD), jnp.float32),
            pltpu.VMEM((_CK,), jnp.int32),
            pltpu.VMEM((_CK,), jnp.int32),
            pltpu.SemaphoreType.DMA,
            pltpu.SemaphoreType.DMA,
        ],
    )
    def k(tbl_hbm, gidx_hbm, sidx_hbm, vals_hbm,
          sunj_hbm, ounj_hbm, sinj_hbm, oinj_hbm,
          iu_hbm, iou_hbm, i2_hbm, io2_hbm,
          out_hbm, o_qu, o_m2a_u, o_m2b_u, o_oa_u, o_ob_u,
          o_qi, o_m2a_i, o_m2b_i, o_oa_i, o_ob_i,
          acc, rows0, rows1, gidx_all, sidx_all, vals_f, zbuf,
          ix0, ix1, sem0, sem1):
        c = lax.axis_index("c")
        s = lax.axis_index("s")
        tbl_c = tbl_hbm.at[c]

        @pl.loop(0, 64)
        def _(i):
            for j in range(8):
                zbuf[i, pl.ds(16 * j, 16)] = jnp.zeros((16,), jnp.float32)

        dnums = lax.GatherDimensionNumbers(
            offset_dims=(), collapsed_slice_dims=(0,), start_index_map=(0,))

        def mul_chunk(jc, rows_b):
            @pl.loop(0, _CK // 16)
            def _(g):
                v16 = vals_f[pl.ds(jc * _CK + g * 16, 16)]
                for l in range(16):
                    vb = lax.gather(
                        v16, jnp.full((16, 1), l, jnp.int32), dnums,
                        slice_sizes=(1,),
                        mode=lax.GatherScatterMode.PROMISE_IN_BOUNDS)
                    i = g * 16 + l
                    for j in range(8):
                        sl = pl.ds(16 * j, 16)
                        rows_b[i, sl] = rows_b[i, sl] * vb

        for t in (0, 1):
            for r in range(4):
                pltpu.sync_copy(zbuf, acc.at[pl.ds(s * 256 + r * 64, 64)])
            plsc.subcore_barrier()

            pltpu.sync_copy(gidx_hbm.at[c, t, s], gidx_all)
            pltpu.sync_copy(sidx_hbm.at[c, t, s], sidx_all)
            pltpu.sync_copy(vals_hbm.at[t, pl.ds(s * _EPT, _EPT)], vals_f)
            pltpu.async_copy(tbl_c.at[gidx_all.at[0]], rows0, sem0)

            def chunk(jc, rows_b, sem_b):
                pltpu.make_async_copy(tbl_c.at[gidx_all.at[jc]], rows_b,
                                      sem_b).wait()
                mul_chunk(jc, rows_b)
                pltpu.sync_copy(rows_b, acc.at[sidx_all.at[jc]], add=True)

            @pl.loop(0, n_chunks, step=2)
            def _(j):
                pltpu.async_copy(tbl_c.at[gidx_all.at[j + 1]], rows1, sem1)
                chunk(j, rows0, sem0)

                @pl.when(j + 2 < n_chunks)
                def _():
                    pltpu.async_copy(tbl_c.at[gidx_all.at[j + 2]], rows0, sem0)
                chunk(j + 1, rows1, sem1)
            plsc.subcore_barrier()

            pltpu.sync_copy(acc.at[pl.ds(s * 256, 256)],
                            out_hbm.at[c, t, pl.ds(s * 256, 256)])
            plsc.subcore_barrier()

        def run_jobs(jobs):
            bufs = ((rows0, ix0, sem0), (rows1, ix1, sem1))

            def start(jj):
                src, idx, _, bpt = jobs[jj]
                buf, ix, sem = bufs[jj % 2]
                pltpu.sync_copy(idx.at[pl.ds(s * bpt, bpt)],
                                ix.at[pl.ds(0, bpt)])
                pltpu.async_copy(src.at[ix.at[pl.ds(0, bpt)]],
                                 buf.at[pl.ds(0, bpt)], sem)

            def finish(jj):
                src, idx, out, bpt = jobs[jj]
                buf, ix, sem = bufs[jj % 2]
                pltpu.make_async_copy(src.at[ix.at[pl.ds(0, bpt)]],
                                      buf.at[pl.ds(0, bpt)], sem).wait()
                pltpu.sync_copy(buf.at[pl.ds(0, bpt)],
                                out.at[pl.ds(s * bpt, bpt)])

            start(0)
            for jj in range(len(jobs)):
                if jj + 1 < len(jobs):
                    start(jj + 1)
                finish(jj)

        @pl.when(c == 0)
        def _():
            run_jobs([(tbl_hbm.at[1], iu_hbm, o_qu, 64),
                      (out_hbm.at[0, 0], iu_hbm, o_m2a_u, 64),
                      (sunj_hbm, iu_hbm, o_m2b_u, 64),
                      (out_hbm.at[0, 1], iou_hbm, o_oa_u, 64),
                      (ounj_hbm, iou_hbm, o_ob_u, 64)])

        @pl.when(c == 1)
        def _():
            run_jobs([(tbl_hbm.at[0], i2_hbm, o_qi, 128),
                      (out_hbm.at[1, 0], i2_hbm, o_m2a_i, 128),
                      (sinj_hbm, i2_hbm, o_m2b_i, 128),
                      (out_hbm.at[1, 1], io2_hbm, o_oa_i, 128),
                      (oinj_hbm, io2_hbm, o_ob_i, 128)])

    return k(tbl_stack, gidx_stack, sidx_stack, vals_stack,
             sample_user_n_j, obs_user_n_j, sample_item_n_j, obs_item_n_j,
             i_users, i_obsu, i_idx2, i_idxo2)


def _fused_side_body(q_ref, k_ref, v_ref, watt_ref, m2a_ref, m2b_ref,
                     wmean_ref, oa_ref, ob_ref, wobs_ref, out_ref):
    w = watt_ref[...]
    m = jnp.dot(w, w.T, preferred_element_type=jnp.float32)
    q = jnp.dot(q_ref[...], m, preferred_element_type=jnp.float32)
    logits = jax.lax.dot_general(q, k_ref[...], (((1,), (1,)), ((), ())),
                                 preferred_element_type=jnp.float32)
    mx = jnp.max(logits, axis=1, keepdims=True)
    p = jnp.exp(logits - mx)
    s = jnp.sum(p, axis=1, keepdims=True)
    att = jnp.dot(p, v_ref[...], preferred_element_type=jnp.float32) / s
    h1 = jnp.dot(att, w, preferred_element_type=jnp.float32)
    h2 = jnp.dot(m2a_ref[...] * m2b_ref[...], wmean_ref[...],
                 preferred_element_type=jnp.float32)
    ho = jnp.tanh(jnp.dot(oa_ref[...] * ob_ref[...], wobs_ref[...],
                          preferred_element_type=jnp.float32))
    h = jnp.tanh(jnp.concatenate([h1, h2, ho], axis=1))
    n = jnp.sqrt(jnp.sum(h * h, axis=1, keepdims=True))
    out_ref[...] = h / jnp.maximum(n, 1e-12)


def _fused_side(q, k, v, watt, m2a, m2b, wmean, oa, ob, wobs):
    bsz = q.shape[0]
    bq = 256
    qmap = lambda i: (i, 0)
    full = lambda i: (0, 0)
    return pl.pallas_call(
        _fused_side_body,
        grid=(bsz // bq,),
        in_specs=[
            pl.BlockSpec((bq, DD), qmap),
            pl.BlockSpec((N_U, DD), full),
            pl.BlockSpec((N_U, DD), full),
            pl.BlockSpec((DD, DD), full),
            pl.BlockSpec((bq, DD), qmap),
            pl.BlockSpec((bq, DD), qmap),
            pl.BlockSpec((DD, DD), full),
            pl.BlockSpec((bq, DD), qmap),
            pl.BlockSpec((bq, DD), qmap),
            pl.BlockSpec((DD, DD), full),
        ],
        out_specs=pl.BlockSpec((bq, 3 * DD), qmap),
        out_shape=jax.ShapeDtypeStruct((bsz, 3 * DD), jnp.float32),
    )(q, k, v, watt, m2a, m2b, wmean, oa, ob, wobs)


def kernel(user_emb, item_emb, W_att_user, W_att_item, W_mean_user,
           W_mean_item, W_obs_user, W_obs_item, sample_user_n_j,
           sample_item_n_j, obs_user_n_j, obs_item_n_j, adj_values,
           obs_adj_values, users, pos_items, neg_items, obs_users,
           obs_pos_items, obs_neg_items, adj_indices, obs_adj_indices):
    tbl_stack = jnp.stack([item_emb, user_emb])
    n_chunks = _EPT // _CK
    gidx_stack = jnp.stack([
        jnp.stack([adj_indices[1], obs_adj_indices[1]]),
        jnp.stack([adj_indices[0], obs_adj_indices[0]]),
    ]).reshape(2, 2, _SC_NS, n_chunks, _CK)
    sidx_stack = jnp.stack([
        jnp.stack([adj_indices[0], obs_adj_indices[0]]),
        jnp.stack([adj_indices[1], obs_adj_indices[1]]),
    ]).reshape(2, 2, _SC_NS, n_chunks, _CK)
    vals_stack = jnp.stack([adj_values, obs_adj_values])
    idx2 = jnp.concatenate([pos_items, neg_items])
    idxo2 = jnp.concatenate([obs_pos_items, obs_neg_items])

    (spmm_o, qu, m2a_u, m2b_u, oa_u, ob_u,
     qi, m2a_i, m2b_i, oa_i, ob_i) = _sc_spmm(
        tbl_stack, gidx_stack, sidx_stack, vals_stack,
        sample_user_n_j, obs_user_n_j, sample_item_n_j, obs_item_n_j,
        users, obs_users, idx2, idxo2)
    spmm_u, spmm_ti = spmm_o[0, 0], spmm_o[1, 0]

    h_u = _fused_side(qu, spmm_ti, item_emb, W_att_user,
                      m2a_u, m2b_u, W_mean_user,
                      oa_u, ob_u, W_obs_user)
    h_pn = _fused_side(qi, spmm_u, user_emb, W_att_item,
                       m2a_i, m2b_i, W_mean_item,
                       oa_i, ob_i, W_obs_item)
    return h_u, h_pn[:1024], h_pn[1024:]


# speedup vs baseline: 10.8687x; 1.0392x over previous
import functools

import jax
import jax.numpy as jnp
from jax import lax
from jax.experimental import pallas as pl
from jax.experimental.pallas import tpu as pltpu
from jax.experimental.pallas import tpu_sc as plsc

N_U = 4096
N_I = 4096
DD = 128
E_EDGES = 131072

_SC_NS = 16
_EPT = E_EDGES // _SC_NS
_CK = 128


def _sc_spmm(tbl_stack, gidx_stack, sidx_stack, vals_stack,
             sample_user_n_j, obs_user_n_j, sample_item_n_j, obs_item_n_j,
             i_users, i_obsu, i_idx2, i_idxo2):
    mesh = plsc.VectorSubcoreMesh(core_axis_name="c", subcore_axis_name="s")

    n_chunks = _EPT // _CK

    f32 = jnp.float32
    gout = lambda n: jax.ShapeDtypeStruct((n, DD), f32)

    @functools.partial(
        pl.kernel,
        out_type=[jax.ShapeDtypeStruct((2, 2, N_U, DD), f32),
                  gout(1024), gout(1024), gout(1024), gout(1024), gout(1024),
                  gout(2048), gout(2048), gout(2048), gout(2048), gout(2048)],
        mesh=mesh,
        compiler_params=pltpu.CompilerParams(needs_layout_passes=False),
        scratch_types=[
            pltpu.VMEM_SHARED((N_U, DD), jnp.float32),
            pltpu.VMEM((_CK, DD), jnp.float32),
            pltpu.VMEM((_CK, DD), jnp.float32),
            pltpu.VMEM((_CK, DD), jnp.float32),
            pltpu.VMEM((_CK, DD), jnp.float32),
            pltpu.VMEM((n_chunks, _CK), jnp.int32),
            pltpu.VMEM((n_chunks, _CK), jnp.int32),
            pltpu.VMEM((_EPT,), jnp.float32),
            pltpu.VMEM((32, DD), jnp.float32),
            pltpu.VMEM((_CK,), jnp.int32),
            pltpu.VMEM((_CK,), jnp.int32),
            pltpu.SemaphoreType.DMA,
            pltpu.SemaphoreType.DMA,
            pltpu.SemaphoreType.DMA,
            pltpu.SemaphoreType.DMA,
            pltpu.SemaphoreType.DMA,
            pltpu.SemaphoreType.DMA,
            pltpu.SemaphoreType.DMA,
            pltpu.SemaphoreType.DMA,
        ],
    )
    def k(tbl_hbm, gidx_hbm, sidx_hbm, vals_hbm,
          sunj_hbm, ounj_hbm, sinj_hbm, oinj_hbm,
          iu_hbm, iou_hbm, i2_hbm, io2_hbm,
          out_hbm, o_qu, o_m2a_u, o_m2b_u, o_oa_u, o_ob_u,
          o_qi, o_m2a_i, o_m2b_i, o_oa_i, o_ob_i,
          acc, rows0, rows1, rows2, rows3, gidx_all, sidx_all, vals_f, zbuf,
          ix0, ix1, gsem0, gsem1, gsem2, gsem3, ssem0, ssem1, ssem2, ssem3):
        c = lax.axis_index("c")
        s = lax.axis_index("s")
        tbl_c = tbl_hbm.at[c]

        @pl.loop(0, 32)
        def _(i):
            for j in range(8):
                zbuf[i, pl.ds(16 * j, 16)] = jnp.zeros((16,), jnp.float32)

        dnums = lax.GatherDimensionNumbers(
            offset_dims=(), collapsed_slice_dims=(0,), start_index_map=(0,))

        def mul_chunk(jc, rows_b):
            @pl.loop(0, _CK // 16)
            def _(g):
                v16 = vals_f[pl.ds(jc * _CK + g * 16, 16)]
                for l in range(16):
                    vb = lax.gather(
                        v16, jnp.full((16, 1), l, jnp.int32), dnums,
                        slice_sizes=(1,),
                        mode=lax.GatherScatterMode.PROMISE_IN_BOUNDS)
                    i = g * 16 + l
                    for j in range(8):
                        sl = pl.ds(16 * j, 16)
                        rows_b[i, sl] = rows_b[i, sl] * vb

        rows = (rows0, rows1, rows2, rows3)
        gsems = (gsem0, gsem1, gsem2, gsem3)
        ssems = (ssem0, ssem1, ssem2, ssem3)

        def g_start(jc, b):
            pltpu.async_copy(tbl_c.at[gidx_all.at[jc]], rows[b], gsems[b])

        def g_wait(jc, b):
            pltpu.make_async_copy(tbl_c.at[gidx_all.at[jc]], rows[b],
                                  gsems[b]).wait()

        def s_start(jc, b):
            pltpu.async_copy(rows[b], acc.at[sidx_all.at[jc]], ssems[b],
                             add=True)

        def s_wait(jc, b):
            pltpu.make_async_copy(rows[b], acc.at[sidx_all.at[jc]],
                                  ssems[b]).wait()

        for t in (0, 1):
            for r in range(8):
                pltpu.sync_copy(zbuf, acc.at[pl.ds(s * 256 + r * 32, 32)])
            plsc.subcore_barrier()

            pltpu.sync_copy(gidx_hbm.at[c, t, s], gidx_all)
            pltpu.sync_copy(sidx_hbm.at[c, t, s], sidx_all)
            pltpu.sync_copy(vals_hbm.at[t, pl.ds(s * _EPT, _EPT)], vals_f)
            g_start(0, 0)
            g_start(1, 1)

            @pl.loop(0, n_chunks, step=4)
            def _(j):
                for u in range(4):
                    jc = j + u
                    bg = (u + 2) % 4

                    @pl.when(jc + 2 < n_chunks)
                    def _(jc=jc, bg=bg, u=u):
                        if u < 2:
                            @pl.when(j > 0)
                            def _():
                                s_wait(jc - 2, bg)
                        else:
                            s_wait(jc - 2, bg)
                        g_start(jc + 2, bg)
                    g_wait(jc, u)
                    mul_chunk(jc, rows[u])
                    s_start(jc, u)
            for u in range(4):
                s_wait(n_chunks - 4 + u, u)
            plsc.subcore_barrier()

            pltpu.sync_copy(acc.at[pl.ds(s * 256, 256)],
                            out_hbm.at[c, t, pl.ds(s * 256, 256)])
            plsc.subcore_barrier()

        def run_jobs(jobs):
            bufs = ((rows0, ix0, gsem0), (rows1, ix1, gsem1))

            def start(jj):
                src, idx, _, bpt = jobs[jj]
                buf, ix, sem = bufs[jj % 2]
                pltpu.sync_copy(idx.at[pl.ds(s * bpt, bpt)],
                                ix.at[pl.ds(0, bpt)])
                pltpu.async_copy(src.at[ix.at[pl.ds(0, bpt)]],
                                 buf.at[pl.ds(0, bpt)], sem)

            def finish(jj):
                src, idx, out, bpt = jobs[jj]
                buf, ix, sem = bufs[jj % 2]
                pltpu.make_async_copy(src.at[ix.at[pl.ds(0, bpt)]],
                                      buf.at[pl.ds(0, bpt)], sem).wait()
                pltpu.sync_copy(buf.at[pl.ds(0, bpt)],
                                out.at[pl.ds(s * bpt, bpt)])

            start(0)
            for jj in range(len(jobs)):
                if jj + 1 < len(jobs):
                    start(jj + 1)
                finish(jj)

        @pl.when(c == 0)
        def _():
            run_jobs([(tbl_hbm.at[1], iu_hbm, o_qu, 64),
                      (out_hbm.at[0, 0], iu_hbm, o_m2a_u, 64),
                      (sunj_hbm, iu_hbm, o_m2b_u, 64),
                      (out_hbm.at[0, 1], iou_hbm, o_oa_u, 64),
                      (ounj_hbm, iou_hbm, o_ob_u, 64)])

        @pl.when(c == 1)
        def _():
            run_jobs([(tbl_hbm.at[0], i2_hbm, o_qi, 128),
                      (out_hbm.at[1, 0], i2_hbm, o_m2a_i, 128),
                      (sinj_hbm, i2_hbm, o_m2b_i, 128),
                      (out_hbm.at[1, 1], io2_hbm, o_oa_i, 128),
                      (oinj_hbm, io2_hbm, o_ob_i, 128)])

    return k(tbl_stack, gidx_stack, sidx_stack, vals_stack,
             sample_user_n_j, obs_user_n_j, sample_item_n_j, obs_item_n_j,
             i_users, i_obsu, i_idx2, i_idxo2)


def _fused_side_body(q_ref, k_ref, v_ref, watt_ref, m2a_ref, m2b_ref,
                     wmean_ref, oa_ref, ob_ref, wobs_ref, out_ref):
    w = watt_ref[...]
    m = jnp.dot(w, w.T, preferred_element_type=jnp.float32)
    q = jnp.dot(q_ref[...], m, preferred_element_type=jnp.float32)
    logits = jax.lax.dot_general(q, k_ref[...], (((1,), (1,)), ((), ())),
                                 preferred_element_type=jnp.float32)
    mx = jnp.max(logits, axis=1, keepdims=True)
    p = jnp.exp(logits - mx)
    s = jnp.sum(p, axis=1, keepdims=True)
    att = jnp.dot(p, v_ref[...], preferred_element_type=jnp.float32) / s
    h1 = jnp.dot(att, w, preferred_element_type=jnp.float32)
    h2 = jnp.dot(m2a_ref[...] * m2b_ref[...], wmean_ref[...],
                 preferred_element_type=jnp.float32)
    ho = jnp.tanh(jnp.dot(oa_ref[...] * ob_ref[...], wobs_ref[...],
                          preferred_element_type=jnp.float32))
    h = jnp.tanh(jnp.concatenate([h1, h2, ho], axis=1))
    n = jnp.sqrt(jnp.sum(h * h, axis=1, keepdims=True))
    out_ref[...] = h / jnp.maximum(n, 1e-12)


def _fused_side(q, k, v, watt, m2a, m2b, wmean, oa, ob, wobs):
    bsz = q.shape[0]
    bq = 256
    qmap = lambda i: (i, 0)
    full = lambda i: (0, 0)
    return pl.pallas_call(
        _fused_side_body,
        grid=(bsz // bq,),
        in_specs=[
            pl.BlockSpec((bq, DD), qmap),
            pl.BlockSpec((N_U, DD), full),
            pl.BlockSpec((N_U, DD), full),
            pl.BlockSpec((DD, DD), full),
            pl.BlockSpec((bq, DD), qmap),
            pl.BlockSpec((bq, DD), qmap),
            pl.BlockSpec((DD, DD), full),
            pl.BlockSpec((bq, DD), qmap),
            pl.BlockSpec((bq, DD), qmap),
            pl.BlockSpec((DD, DD), full),
        ],
        out_specs=pl.BlockSpec((bq, 3 * DD), qmap),
        out_shape=jax.ShapeDtypeStruct((bsz, 3 * DD), jnp.float32),
    )(q, k, v, watt, m2a, m2b, wmean, oa, ob, wobs)


def kernel(user_emb, item_emb, W_att_user, W_att_item, W_mean_user,
           W_mean_item, W_obs_user, W_obs_item, sample_user_n_j,
           sample_item_n_j, obs_user_n_j, obs_item_n_j, adj_values,
           obs_adj_values, users, pos_items, neg_items, obs_users,
           obs_pos_items, obs_neg_items, adj_indices, obs_adj_indices):
    tbl_stack = jnp.stack([item_emb, user_emb])
    n_chunks = _EPT // _CK
    gidx_stack = jnp.stack([
        jnp.stack([adj_indices[1], obs_adj_indices[1]]),
        jnp.stack([adj_indices[0], obs_adj_indices[0]]),
    ]).reshape(2, 2, _SC_NS, n_chunks, _CK)
    sidx_stack = jnp.stack([
        jnp.stack([adj_indices[0], obs_adj_indices[0]]),
        jnp.stack([adj_indices[1], obs_adj_indices[1]]),
    ]).reshape(2, 2, _SC_NS, n_chunks, _CK)
    vals_stack = jnp.stack([adj_values, obs_adj_values])
    idx2 = jnp.concatenate([pos_items, neg_items])
    idxo2 = jnp.concatenate([obs_pos_items, obs_neg_items])

    (spmm_o, qu, m2a_u, m2b_u, oa_u, ob_u,
     qi, m2a_i, m2b_i, oa_i, ob_i) = _sc_spmm(
        tbl_stack, gidx_stack, sidx_stack, vals_stack,
        sample_user_n_j, obs_user_n_j, sample_item_n_j, obs_item_n_j,
        users, obs_users, idx2, idxo2)
    spmm_u, spmm_ti = spmm_o[0, 0], spmm_o[1, 0]

    h_u = _fused_side(qu, spmm_ti, item_emb, W_att_user,
                      m2a_u, m2b_u, W_mean_user,
                      oa_u, ob_u, W_obs_user)
    h_pn = _fused_side(qi, spmm_u, user_emb, W_att_item,
                       m2a_i, m2b_i, W_mean_item,
                       oa_i, ob_i, W_obs_item)
    return h_u, h_pn[:1024], h_pn[1024:]


# speedup vs baseline: 11.9597x; 1.1004x over previous
import functools

import jax
import jax.numpy as jnp
from jax import lax
from jax.experimental import pallas as pl
from jax.experimental.pallas import tpu as pltpu
from jax.experimental.pallas import tpu_sc as plsc

N_U = 4096
N_I = 4096
DD = 128
E_EDGES = 131072

_SC_NS = 16
_EPT = E_EDGES // _SC_NS
_CK = 128
_NCH = _EPT // _CK


def _sc_phase(tbl_stack, gidx, sidx, vals, aux0, aux1, idx0, idx1, with_emb):
    f32 = jnp.float32
    nj = 3 if with_emb else 2
    outs = [jax.ShapeDtypeStruct((2, N_U, DD), f32)]
    outs += [jax.ShapeDtypeStruct((1024, DD), f32)] * nj
    outs += [jax.ShapeDtypeStruct((2048, DD), f32)] * nj
    mesh = plsc.VectorSubcoreMesh(core_axis_name="c", subcore_axis_name="s")

    @functools.partial(
        pl.kernel,
        out_type=outs,
        mesh=mesh,
        compiler_params=pltpu.CompilerParams(needs_layout_passes=False),
        scratch_types=[
            pltpu.VMEM_SHARED((N_U, DD), f32),
            pltpu.VMEM((_CK, DD), f32),
            pltpu.VMEM((_CK, DD), f32),
            pltpu.VMEM((_CK, DD), f32),
            pltpu.VMEM((_CK, DD), f32),
            pltpu.VMEM((_NCH, _CK), jnp.int32),
            pltpu.VMEM((_NCH, _CK), jnp.int32),
            pltpu.VMEM((_EPT,), f32),
            pltpu.VMEM((32, DD), f32),
            pltpu.VMEM((_CK,), jnp.int32),
            pltpu.VMEM((_CK,), jnp.int32),
            pltpu.SemaphoreType.DMA,
            pltpu.SemaphoreType.DMA,
            pltpu.SemaphoreType.DMA,
            pltpu.SemaphoreType.DMA,
            pltpu.SemaphoreType.DMA,
            pltpu.SemaphoreType.DMA,
            pltpu.SemaphoreType.DMA,
            pltpu.SemaphoreType.DMA,
        ],
    )
    def k(tbl_hbm, gidx_hbm, sidx_hbm, vals_hbm, aux0_hbm, aux1_hbm,
          idx0_hbm, idx1_hbm, out_hbm, *rest):
        gouts = rest[:2 * nj]
        (acc, rows0, rows1, rows2, rows3, gidx_all, sidx_all, vals_f, zbuf,
         ix0, ix1, gsem0, gsem1, gsem2, gsem3,
         ssem0, ssem1, ssem2, ssem3) = rest[2 * nj:]
        c = lax.axis_index("c")
        s = lax.axis_index("s")
        tbl_c = tbl_hbm.at[c]

        @pl.loop(0, 32)
        def _(i):
            for j in range(8):
                zbuf[i, pl.ds(16 * j, 16)] = jnp.zeros((16,), f32)

        dnums = lax.GatherDimensionNumbers(
            offset_dims=(), collapsed_slice_dims=(0,), start_index_map=(0,))

        def mul_chunk(jc, rows_b):
            @pl.loop(0, _CK // 16)
            def _(g):
                v16 = vals_f[pl.ds(jc * _CK + g * 16, 16)]
                for l in range(16):
                    vb = lax.gather(
                        v16, jnp.full((16, 1), l, jnp.int32), dnums,
                        slice_sizes=(1,),
                        mode=lax.GatherScatterMode.PROMISE_IN_BOUNDS)
                    i = g * 16 + l
                    for j in range(8):
                        sl = pl.ds(16 * j, 16)
                        rows_b[i, sl] = rows_b[i, sl] * vb

        rows = (rows0, rows1, rows2, rows3)
        gsems = (gsem0, gsem1, gsem2, gsem3)
        ssems = (ssem0, ssem1, ssem2, ssem3)

        def g_start(jc, b):
            pltpu.async_copy(tbl_c.at[gidx_all.at[jc]], rows[b], gsems[b])

        def g_wait(jc, b):
            pltpu.make_async_copy(tbl_c.at[gidx_all.at[jc]], rows[b],
                                  gsems[b]).wait()

        def s_start(jc, b):
            pltpu.async_copy(rows[b], acc.at[sidx_all.at[jc]], ssems[b],
                             add=True)

        def s_wait(jc, b):
            pltpu.make_async_copy(rows[b], acc.at[sidx_all.at[jc]],
                                  ssems[b]).wait()

        for r in range(8):
            pltpu.sync_copy(zbuf, acc.at[pl.ds(s * 256 + r * 32, 32)])
        plsc.subcore_barrier()

        pltpu.sync_copy(gidx_hbm.at[c, s], gidx_all)
        pltpu.sync_copy(sidx_hbm.at[c, s], sidx_all)
        pltpu.sync_copy(vals_hbm.at[pl.ds(s * _EPT, _EPT)], vals_f)
        g_start(0, 0)
        g_start(1, 1)

        @pl.loop(0, _NCH, step=4)
        def _(j):
            for u in range(4):
                jc = j + u
                bg = (u + 2) % 4

                @pl.when(jc + 2 < _NCH)
                def _(jc=jc, bg=bg, u=u):
                    if u < 2:
                        @pl.when(j > 0)
                        def _():
                            s_wait(jc - 2, bg)
                    else:
                        s_wait(jc - 2, bg)
                    g_start(jc + 2, bg)
                g_wait(jc, u)
                mul_chunk(jc, rows[u])
                s_start(jc, u)
        for u in range(4):
            s_wait(_NCH - 4 + u, u)
        plsc.subcore_barrier()

        pltpu.sync_copy(acc.at[pl.ds(s * 256, 256)],
                        out_hbm.at[c, pl.ds(s * 256, 256)])
        plsc.subcore_barrier()

        def run_jobs(jobs):
            bufs = ((rows0, ix0, gsem0), (rows1, ix1, gsem1))

            def start(jj):
                src, idx, _, bpt = jobs[jj]
                buf, ix, sem = bufs[jj % 2]
                pltpu.sync_copy(idx.at[pl.ds(s * bpt, bpt)],
                                ix.at[pl.ds(0, bpt)])
                pltpu.async_copy(src.at[ix.at[pl.ds(0, bpt)]],
                                 buf.at[pl.ds(0, bpt)], sem)

            def finish(jj):
                src, idx, out, bpt = jobs[jj]
                buf, ix, sem = bufs[jj % 2]
                pltpu.make_async_copy(src.at[ix.at[pl.ds(0, bpt)]],
                                      buf.at[pl.ds(0, bpt)], sem).wait()
                pltpu.sync_copy(buf.at[pl.ds(0, bpt)],
                                out.at[pl.ds(s * bpt, bpt)])

            start(0)
            for jj in range(len(jobs)):
                if jj + 1 < len(jobs):
                    start(jj + 1)
                finish(jj)

        @pl.when(c == 0)
        def _():
            jobs = [(out_hbm.at[0], idx0_hbm, gouts[nj - 2], 64),
                    (aux0_hbm, idx0_hbm, gouts[nj - 1], 64)]
            if with_emb:
                jobs.insert(0, (tbl_hbm.at[1], idx0_hbm, gouts[0], 64))
            run_jobs(jobs)

        @pl.when(c == 1)
        def _():
            jobs = [(out_hbm.at[1], idx1_hbm, gouts[2 * nj - 2], 128),
                    (aux1_hbm, idx1_hbm, gouts[2 * nj - 1], 128)]
            if with_emb:
                jobs.insert(0, (tbl_hbm.at[0], idx1_hbm, gouts[nj], 128))
            run_jobs(jobs)

    return k(tbl_stack, gidx, sidx, vals, aux0, aux1, idx0, idx1)


def _side_pre_body(q_ref, k_ref, v_ref, watt_ref, m2a_ref, m2b_ref,
                   wmean_ref, out_ref):
    w = watt_ref[...]
    m = jnp.dot(w, w.T, preferred_element_type=jnp.float32)
    q = jnp.dot(q_ref[...], m, preferred_element_type=jnp.float32)
    logits = jax.lax.dot_general(q, k_ref[...], (((1,), (1,)), ((), ())),
                                 preferred_element_type=jnp.float32)
    mx = jnp.max(logits, axis=1, keepdims=True)
    p = jnp.exp(logits - mx)
    sm = jnp.sum(p, axis=1, keepdims=True)
    att = jnp.dot(p, v_ref[...], preferred_element_type=jnp.float32) / sm
    h1 = jnp.dot(att, w, preferred_element_type=jnp.float32)
    h2 = jnp.dot(m2a_ref[...] * m2b_ref[...], wmean_ref[...],
                 preferred_element_type=jnp.float32)
    out_ref[...] = jnp.concatenate([h1, h2], axis=1)


def _side_pre(q, k, v, watt, m2a, m2b, wmean):
    bsz = q.shape[0]
    bq = 256
    qmap = lambda i: (i, 0)
    full = lambda i: (0, 0)
    return pl.pallas_call(
        _side_pre_body,
        grid=(bsz // bq,),
        in_specs=[
            pl.BlockSpec((bq, DD), qmap),
            pl.BlockSpec((N_U, DD), full),
            pl.BlockSpec((N_U, DD), full),
            pl.BlockSpec((DD, DD), full),
            pl.BlockSpec((bq, DD), qmap),
            pl.BlockSpec((bq, DD), qmap),
            pl.BlockSpec((DD, DD), full),
        ],
        out_specs=pl.BlockSpec((bq, 2 * DD), qmap),
        out_shape=jax.ShapeDtypeStruct((bsz, 2 * DD), jnp.float32),
    )(q, k, v, watt, m2a, m2b, wmean)


def _final_body(hs_ref, oa_ref, ob_ref, wobs_ref, out_ref):
    ho = jnp.tanh(jnp.dot(oa_ref[...] * ob_ref[...], wobs_ref[...],
                          preferred_element_type=jnp.float32))
    h = jnp.tanh(jnp.concatenate([hs_ref[...], ho], axis=1))
    n = jnp.sqrt(jnp.sum(h * h, axis=1, keepdims=True))
    out_ref[...] = h / jnp.maximum(n, 1e-12)


def _final(hs, oa, ob, wobs):
    bsz = hs.shape[0]
    bq = 256
    qmap = lambda i: (i, 0)
    full = lambda i: (0, 0)
    return pl.pallas_call(
        _final_body,
        grid=(bsz // bq,),
        in_specs=[
            pl.BlockSpec((bq, 2 * DD), qmap),
            pl.BlockSpec((bq, DD), qmap),
            pl.BlockSpec((bq, DD), qmap),
            pl.BlockSpec((DD, DD), full),
        ],
        out_specs=pl.BlockSpec((bq, 3 * DD), qmap),
        out_shape=jax.ShapeDtypeStruct((bsz, 3 * DD), jnp.float32),
    )(hs, oa, ob, wobs)


def kernel(user_emb, item_emb, W_att_user, W_att_item, W_mean_user,
           W_mean_item, W_obs_user, W_obs_item, sample_user_n_j,
           sample_item_n_j, obs_user_n_j, obs_item_n_j, adj_values,
           obs_adj_values, users, pos_items, neg_items, obs_users,
           obs_pos_items, obs_neg_items, adj_indices, obs_adj_indices):
    tbl_stack = jnp.stack([item_emb, user_emb])
    rs = lambda x: x.reshape(_SC_NS, _NCH, _CK)
    gidx1 = jnp.stack([rs(adj_indices[1]), rs(adj_indices[0])])
    sidx1 = jnp.stack([rs(adj_indices[0]), rs(adj_indices[1])])
    gidx2 = jnp.stack([rs(obs_adj_indices[1]), rs(obs_adj_indices[0])])
    sidx2 = jnp.stack([rs(obs_adj_indices[0]), rs(obs_adj_indices[1])])
    idx2 = jnp.concatenate([pos_items, neg_items])
    idxo2 = jnp.concatenate([obs_pos_items, obs_neg_items])

    spmm_o, qu, m2a_u, m2b_u, qi, m2a_i, m2b_i = _sc_phase(
        tbl_stack, gidx1, sidx1, adj_values,
        sample_user_n_j, sample_item_n_j, users, idx2, True)
    _, oa_u, ob_u, oa_i, ob_i = _sc_phase(
        tbl_stack, gidx2, sidx2, obs_adj_values,
        obs_user_n_j, obs_item_n_j, obs_users, idxo2, False)

    hs_u = _side_pre(qu, spmm_o[1], item_emb, W_att_user,
                     m2a_u, m2b_u, W_mean_user)
    hs_pn = _side_pre(qi, spmm_o[0], user_emb, W_att_item,
                      m2a_i, m2b_i, W_mean_item)
    h_u = _final(hs_u, oa_u, ob_u, W_obs_user)
    h_pn = _final(hs_pn, oa_i, ob_i, W_obs_item)
    return h_u, h_pn[:1024], h_pn[1024:]


# speedup vs baseline: 12.1441x; 1.0154x over previous
import functools

import jax
import jax.numpy as jnp
from jax import lax
from jax.experimental import pallas as pl
from jax.experimental.pallas import tpu as pltpu
from jax.experimental.pallas import tpu_sc as plsc

N_U = 4096
N_I = 4096
DD = 128
E_EDGES = 131072

_SC_NS = 16
_EPT = E_EDGES // _SC_NS
_CK = 128
_NCH = _EPT // _CK


def _sc_phase(tbl_stack, gidx, sidx, vals, aux0, aux1, idx0, idx1, with_emb):
    f32 = jnp.float32
    nj = 3 if with_emb else 2
    outs = [jax.ShapeDtypeStruct((2, N_U, DD), f32)]
    outs += [jax.ShapeDtypeStruct((1024, DD), f32)] * nj
    outs += [jax.ShapeDtypeStruct((2048, DD), f32)] * nj
    mesh = plsc.VectorSubcoreMesh(core_axis_name="c", subcore_axis_name="s")

    @functools.partial(
        pl.kernel,
        out_type=outs,
        mesh=mesh,
        compiler_params=pltpu.CompilerParams(needs_layout_passes=False),
        scratch_types=[
            pltpu.VMEM_SHARED((N_U, DD), f32),
            pltpu.VMEM((_CK, DD), f32),
            pltpu.VMEM((_CK, DD), f32),
            pltpu.VMEM((_CK, DD), f32),
            pltpu.VMEM((_CK, DD), f32),
            pltpu.VMEM((_NCH, _CK), jnp.int32),
            pltpu.VMEM((_NCH, _CK), jnp.int32),
            pltpu.VMEM((_EPT,), f32),
            pltpu.VMEM((32, DD), f32),
            pltpu.VMEM((_CK,), jnp.int32),
            pltpu.VMEM((_CK,), jnp.int32),
            pltpu.SemaphoreType.DMA,
            pltpu.SemaphoreType.DMA,
            pltpu.SemaphoreType.DMA,
            pltpu.SemaphoreType.DMA,
            pltpu.SemaphoreType.DMA,
            pltpu.SemaphoreType.DMA,
            pltpu.SemaphoreType.DMA,
            pltpu.SemaphoreType.DMA,
        ],
    )
    def k(tbl_hbm, gidx_hbm, sidx_hbm, vals_hbm, aux0_hbm, aux1_hbm,
          idx0_hbm, idx1_hbm, out_hbm, *rest):
        gouts = rest[:2 * nj]
        (acc, rows0, rows1, rows2, rows3, gidx_all, sidx_all, vals_f, zbuf,
         ix0, ix1, gsem0, gsem1, gsem2, gsem3,
         ssem0, ssem1, ssem2, ssem3) = rest[2 * nj:]
        c = lax.axis_index("c")
        s = lax.axis_index("s")
        tbl_c = tbl_hbm.at[c]

        @pl.loop(0, 32)
        def _(i):
            for j in range(8):
                zbuf[i, pl.ds(16 * j, 16)] = jnp.zeros((16,), f32)

        dnums = lax.GatherDimensionNumbers(
            offset_dims=(), collapsed_slice_dims=(0,), start_index_map=(0,))

        def mul_chunk(jc, rows_b):
            @pl.loop(0, _CK // 16)
            def _(g):
                v16 = vals_f[pl.ds(jc * _CK + g * 16, 16)]
                for l in range(16):
                    vb = lax.gather(
                        v16, jnp.full((16, 1), l, jnp.int32), dnums,
                        slice_sizes=(1,),
                        mode=lax.GatherScatterMode.PROMISE_IN_BOUNDS)
                    i = g * 16 + l
                    for j in range(8):
                        sl = pl.ds(16 * j, 16)
                        rows_b[i, sl] = rows_b[i, sl] * vb

        rows = (rows0, rows1, rows2, rows3)
        gsems = (gsem0, gsem1, gsem2, gsem3)
        ssems = (ssem0, ssem1, ssem2, ssem3)

        def g_start(jc, b):
            pltpu.async_copy(tbl_c.at[gidx_all.at[jc]], rows[b], gsems[b])

        def g_wait(jc, b):
            pltpu.make_async_copy(tbl_c.at[gidx_all.at[jc]], rows[b],
                                  gsems[b]).wait()

        def s_start(jc, b):
            pltpu.async_copy(rows[b], acc.at[sidx_all.at[jc]], ssems[b],
                             add=True)

        def s_wait(jc, b):
            pltpu.make_async_copy(rows[b], acc.at[sidx_all.at[jc]],
                                  ssems[b]).wait()

        for r in range(8):
            pltpu.sync_copy(zbuf, acc.at[pl.ds(s * 256 + r * 32, 32)])
        plsc.subcore_barrier()

        pltpu.sync_copy(gidx_hbm.at[c, s], gidx_all)
        pltpu.sync_copy(sidx_hbm.at[c, s], sidx_all)
        pltpu.sync_copy(vals_hbm.at[pl.ds(s * _EPT, _EPT)], vals_f)
        g_start(0, 0)
        g_start(1, 1)

        @pl.loop(0, _NCH, step=4)
        def _(j):
            for u in range(4):
                jc = j + u
                bg = (u + 2) % 4

                @pl.when(jc + 2 < _NCH)
                def _(jc=jc, bg=bg, u=u):
                    if u < 2:
                        @pl.when(j > 0)
                        def _():
                            s_wait(jc - 2, bg)
                    else:
                        s_wait(jc - 2, bg)
                    g_start(jc + 2, bg)
                g_wait(jc, u)
                mul_chunk(jc, rows[u])
                s_start(jc, u)
        for u in range(4):
            s_wait(_NCH - 4 + u, u)
        plsc.subcore_barrier()

        pltpu.sync_copy(acc.at[pl.ds(s * 256, 256)],
                        out_hbm.at[c, pl.ds(s * 256, 256)])
        plsc.subcore_barrier()

        def run_jobs(jobs):
            bufs = ((rows0, ix0, gsem0), (rows1, ix1, gsem1))

            def start(jj):
                src, idx, _, bpt = jobs[jj]
                buf, ix, sem = bufs[jj % 2]
                pltpu.sync_copy(idx.at[pl.ds(s * bpt, bpt)],
                                ix.at[pl.ds(0, bpt)])
                pltpu.async_copy(src.at[ix.at[pl.ds(0, bpt)]],
                                 buf.at[pl.ds(0, bpt)], sem)

            def finish(jj):
                src, idx, out, bpt = jobs[jj]
                buf, ix, sem = bufs[jj % 2]
                pltpu.make_async_copy(src.at[ix.at[pl.ds(0, bpt)]],
                                      buf.at[pl.ds(0, bpt)], sem).wait()
                pltpu.sync_copy(buf.at[pl.ds(0, bpt)],
                                out.at[pl.ds(s * bpt, bpt)])

            start(0)
            for jj in range(len(jobs)):
                if jj + 1 < len(jobs):
                    start(jj + 1)
                finish(jj)

        @pl.when(c == 0)
        def _():
            jobs = [(out_hbm.at[0], idx0_hbm, gouts[nj - 2], 64),
                    (aux0_hbm, idx0_hbm, gouts[nj - 1], 64)]
            if with_emb:
                jobs.insert(0, (tbl_hbm.at[1], idx0_hbm, gouts[0], 64))
            run_jobs(jobs)

        @pl.when(c == 1)
        def _():
            jobs = [(out_hbm.at[1], idx1_hbm, gouts[2 * nj - 2], 128),
                    (aux1_hbm, idx1_hbm, gouts[2 * nj - 1], 128)]
            if with_emb:
                jobs.insert(0, (tbl_hbm.at[0], idx1_hbm, gouts[nj], 128))
            run_jobs(jobs)

    return k(tbl_stack, gidx, sidx, vals, aux0, aux1, idx0, idx1)


def _side_pre_body(q_ref, k_ref, v_ref, watt_ref, m2a_ref, m2b_ref,
                   wmean_ref, out_ref):
    w = watt_ref[...]
    m = jnp.dot(w, w.T, preferred_element_type=jnp.float32)
    q = jnp.dot(q_ref[...], m, preferred_element_type=jnp.float32)
    logits = jax.lax.dot_general(
        q.astype(jnp.bfloat16), k_ref[...].astype(jnp.bfloat16),
        (((1,), (1,)), ((), ())), preferred_element_type=jnp.float32)
    mx = jnp.max(logits, axis=1, keepdims=True)
    p = jnp.exp(logits - mx)
    sm = jnp.sum(p, axis=1, keepdims=True)
    att = jnp.dot(p.astype(jnp.bfloat16), v_ref[...].astype(jnp.bfloat16),
                  preferred_element_type=jnp.float32) / sm
    h1 = jnp.dot(att, w, preferred_element_type=jnp.float32)
    h2 = jnp.dot(m2a_ref[...] * m2b_ref[...], wmean_ref[...],
                 preferred_element_type=jnp.float32)
    out_ref[...] = jnp.concatenate([h1, h2], axis=1)


def _side_pre(q, k, v, watt, m2a, m2b, wmean):
    bsz = q.shape[0]
    bq = 256
    qmap = lambda i: (i, 0)
    full = lambda i: (0, 0)
    return pl.pallas_call(
        _side_pre_body,
        grid=(bsz // bq,),
        in_specs=[
            pl.BlockSpec((bq, DD), qmap),
            pl.BlockSpec((N_U, DD), full),
            pl.BlockSpec((N_U, DD), full),
            pl.BlockSpec((DD, DD), full),
            pl.BlockSpec((bq, DD), qmap),
            pl.BlockSpec((bq, DD), qmap),
            pl.BlockSpec((DD, DD), full),
        ],
        out_specs=pl.BlockSpec((bq, 2 * DD), qmap),
        out_shape=jax.ShapeDtypeStruct((bsz, 2 * DD), jnp.float32),
    )(q, k, v, watt, m2a, m2b, wmean)


def _final_body(hs_ref, oa_ref, ob_ref, wobs_ref, out_ref):
    ho = jnp.tanh(jnp.dot(oa_ref[...] * ob_ref[...], wobs_ref[...],
                          preferred_element_type=jnp.float32))
    h = jnp.tanh(jnp.concatenate([hs_ref[...], ho], axis=1))
    n = jnp.sqrt(jnp.sum(h * h, axis=1, keepdims=True))
    out_ref[...] = h / jnp.maximum(n, 1e-12)


def _final(hs, oa, ob, wobs):
    bsz = hs.shape[0]
    bq = 256
    qmap = lambda i: (i, 0)
    full = lambda i: (0, 0)
    return pl.pallas_call(
        _final_body,
        grid=(bsz // bq,),
        in_specs=[
            pl.BlockSpec((bq, 2 * DD), qmap),
            pl.BlockSpec((bq, DD), qmap),
            pl.BlockSpec((bq, DD), qmap),
            pl.BlockSpec((DD, DD), full),
        ],
        out_specs=pl.BlockSpec((bq, 3 * DD), qmap),
        out_shape=jax.ShapeDtypeStruct((bsz, 3 * DD), jnp.float32),
    )(hs, oa, ob, wobs)


def kernel(user_emb, item_emb, W_att_user, W_att_item, W_mean_user,
           W_mean_item, W_obs_user, W_obs_item, sample_user_n_j,
           sample_item_n_j, obs_user_n_j, obs_item_n_j, adj_values,
           obs_adj_values, users, pos_items, neg_items, obs_users,
           obs_pos_items, obs_neg_items, adj_indices, obs_adj_indices):
    tbl_stack = jnp.stack([item_emb, user_emb])
    rs = lambda x: x.reshape(_SC_NS, _NCH, _CK)
    gidx1 = jnp.stack([rs(adj_indices[1]), rs(adj_indices[0])])
    sidx1 = jnp.stack([rs(adj_indices[0]), rs(adj_indices[1])])
    gidx2 = jnp.stack([rs(obs_adj_indices[1]), rs(obs_adj_indices[0])])
    sidx2 = jnp.stack([rs(obs_adj_indices[0]), rs(obs_adj_indices[1])])
    idx2 = jnp.concatenate([pos_items, neg_items])
    idxo2 = jnp.concatenate([obs_pos_items, obs_neg_items])

    spmm_o, qu, m2a_u, m2b_u, qi, m2a_i, m2b_i = _sc_phase(
        tbl_stack, gidx1, sidx1, adj_values,
        sample_user_n_j, sample_item_n_j, users, idx2, True)
    _, oa_u, ob_u, oa_i, ob_i = _sc_phase(
        tbl_stack, gidx2, sidx2, obs_adj_values,
        obs_user_n_j, obs_item_n_j, obs_users, idxo2, False)

    hs_u = _side_pre(qu, spmm_o[1], item_emb, W_att_user,
                     m2a_u, m2b_u, W_mean_user)
    hs_pn = _side_pre(qi, spmm_o[0], user_emb, W_att_item,
                      m2a_i, m2b_i, W_mean_item)
    h_u = _final(hs_u, oa_u, ob_u, W_obs_user)
    h_pn = _final(hs_pn, oa_i, ob_i, W_obs_item)
    return h_u, h_pn[:1024], h_pn[1024:]


# speedup vs baseline: 14.5664x; 1.1995x over previous
import functools

import jax
import jax.numpy as jnp
from jax import lax
from jax.experimental import pallas as pl
from jax.experimental.pallas import tpu as pltpu
from jax.experimental.pallas import tpu_sc as plsc

N_U = 4096
N_I = 4096
DD = 128
E_EDGES = 131072

_SC_NS = 16
_EPT = E_EDGES // _SC_NS
_CK = 128
_NCH = _EPT // _CK


def _sc_phase(tbl_stack, gidx, sidx, vals, aux0, aux1, idx0, idx1, with_emb):
    f32 = jnp.float32
    nj = 3 if with_emb else 2
    outs = [jax.ShapeDtypeStruct((2, N_U, DD), f32)]
    outs += [jax.ShapeDtypeStruct((1024, DD), f32)] * nj
    outs += [jax.ShapeDtypeStruct((2048, DD), f32)] * nj
    mesh = plsc.VectorSubcoreMesh(core_axis_name="c", subcore_axis_name="s")

    @functools.partial(
        pl.kernel,
        out_type=outs,
        mesh=mesh,
        compiler_params=pltpu.CompilerParams(needs_layout_passes=False),
        scratch_types=[
            pltpu.VMEM_SHARED((N_U, DD), f32),
            pltpu.VMEM((_CK, DD), f32),
            pltpu.VMEM((_CK, DD), f32),
            pltpu.VMEM((_CK, DD), f32),
            pltpu.VMEM((_CK, DD), f32),
            pltpu.VMEM((_NCH, _CK), jnp.int32),
            pltpu.VMEM((_NCH, _CK), jnp.int32),
            pltpu.VMEM((_EPT,), f32),
            pltpu.VMEM((32, DD), f32),
            pltpu.VMEM((_CK,), jnp.int32),
            pltpu.VMEM((_CK,), jnp.int32),
            pltpu.SemaphoreType.DMA,
            pltpu.SemaphoreType.DMA,
            pltpu.SemaphoreType.DMA,
            pltpu.SemaphoreType.DMA,
            pltpu.SemaphoreType.DMA,
            pltpu.SemaphoreType.DMA,
            pltpu.SemaphoreType.DMA,
            pltpu.SemaphoreType.DMA,
        ],
    )
    def k(tbl_hbm, gidx_hbm, sidx_hbm, vals_hbm, aux0_hbm, aux1_hbm,
          idx0_hbm, idx1_hbm, out_hbm, *rest):
        gouts = rest[:2 * nj]
        (acc, rows0, rows1, rows2, rows3, gidx_all, sidx_all, vals_f, zbuf,
         ix0, ix1, gsem0, gsem1, gsem2, gsem3,
         ssem0, ssem1, ssem2, ssem3) = rest[2 * nj:]
        c = lax.axis_index("c")
        s = lax.axis_index("s")
        tbl_c = tbl_hbm.at[c]

        @pl.loop(0, 32)
        def _(i):
            for j in range(8):
                zbuf[i, pl.ds(16 * j, 16)] = jnp.zeros((16,), f32)

        dnums = lax.GatherDimensionNumbers(
            offset_dims=(), collapsed_slice_dims=(0,), start_index_map=(0,))

        def mul_chunk(jc, rows_b):
            @functools.partial(plsc.parallel_loop, 0, _CK // 16, unroll=2)
            def _(g):
                v16 = vals_f[pl.ds(jc * _CK + g * 16, 16)]
                for l in range(16):
                    vb = lax.gather(
                        v16, jnp.full((16, 1), l, jnp.int32), dnums,
                        slice_sizes=(1,),
                        mode=lax.GatherScatterMode.PROMISE_IN_BOUNDS)
                    i = g * 16 + l
                    for j in range(8):
                        sl = pl.ds(16 * j, 16)
                        rows_b[i, sl] = rows_b[i, sl] * vb

        rows = (rows0, rows1, rows2, rows3)
        gsems = (gsem0, gsem1, gsem2, gsem3)
        ssems = (ssem0, ssem1, ssem2, ssem3)

        def g_start(jc, b):
            pltpu.async_copy(tbl_c.at[gidx_all.at[jc]], rows[b], gsems[b])

        def g_wait(jc, b):
            pltpu.make_async_copy(tbl_c.at[gidx_all.at[jc]], rows[b],
                                  gsems[b]).wait()

        def s_start(jc, b):
            pltpu.async_copy(rows[b], acc.at[sidx_all.at[jc]], ssems[b],
                             add=True)

        def s_wait(jc, b):
            pltpu.make_async_copy(rows[b], acc.at[sidx_all.at[jc]],
                                  ssems[b]).wait()

        for r in range(8):
            pltpu.async_copy(zbuf, acc.at[pl.ds(s * 256 + r * 32, 32)],
                             gsems[r % 4])
        pltpu.async_copy(gidx_hbm.at[c, s], gidx_all, ssems[0])
        pltpu.async_copy(sidx_hbm.at[c, s], sidx_all, ssems[1])
        pltpu.async_copy(vals_hbm.at[pl.ds(s * _EPT, _EPT)], vals_f, ssems[2])
        for r in range(8):
            pltpu.make_async_copy(zbuf, acc.at[pl.ds(s * 256 + r * 32, 32)],
                                  gsems[r % 4]).wait()
        pltpu.make_async_copy(gidx_hbm.at[c, s], gidx_all, ssems[0]).wait()
        pltpu.make_async_copy(sidx_hbm.at[c, s], sidx_all, ssems[1]).wait()
        pltpu.make_async_copy(vals_hbm.at[pl.ds(s * _EPT, _EPT)], vals_f,
                              ssems[2]).wait()
        plsc.subcore_barrier()
        g_start(0, 0)
        g_start(1, 1)

        @pl.loop(0, _NCH, step=4)
        def _(j):
            for u in range(4):
                jc = j + u
                bg = (u + 2) % 4

                @pl.when(jc + 2 < _NCH)
                def _(jc=jc, bg=bg, u=u):
                    if u < 2:
                        @pl.when(j > 0)
                        def _():
                            s_wait(jc - 2, bg)
                    else:
                        s_wait(jc - 2, bg)
                    g_start(jc + 2, bg)
                g_wait(jc, u)
                mul_chunk(jc, rows[u])
                s_start(jc, u)
        for u in range(4):
            s_wait(_NCH - 4 + u, u)
        plsc.subcore_barrier()

        pltpu.sync_copy(acc.at[pl.ds(s * 256, 256)],
                        out_hbm.at[c, pl.ds(s * 256, 256)])
        plsc.subcore_barrier()

        def run_jobs(jobs):
            bufs = ((rows0, ix0, gsem0), (rows1, ix1, gsem1))

            def start(jj):
                src, idx, _, bpt = jobs[jj]
                buf, ix, sem = bufs[jj % 2]
                pltpu.sync_copy(idx.at[pl.ds(s * bpt, bpt)],
                                ix.at[pl.ds(0, bpt)])
                pltpu.async_copy(src.at[ix.at[pl.ds(0, bpt)]],
                                 buf.at[pl.ds(0, bpt)], sem)

            def finish(jj):
                src, idx, out, bpt = jobs[jj]
                buf, ix, sem = bufs[jj % 2]
                pltpu.make_async_copy(src.at[ix.at[pl.ds(0, bpt)]],
                                      buf.at[pl.ds(0, bpt)], sem).wait()
                pltpu.sync_copy(buf.at[pl.ds(0, bpt)],
                                out.at[pl.ds(s * bpt, bpt)])

            start(0)
            for jj in range(len(jobs)):
                if jj + 1 < len(jobs):
                    start(jj + 1)
                finish(jj)

        @pl.when(c == 0)
        def _():
            jobs = [(out_hbm.at[0], idx0_hbm, gouts[nj - 2], 64),
                    (aux0_hbm, idx0_hbm, gouts[nj - 1], 64)]
            if with_emb:
                jobs.insert(0, (tbl_hbm.at[1], idx0_hbm, gouts[0], 64))
            run_jobs(jobs)

        @pl.when(c == 1)
        def _():
            jobs = [(out_hbm.at[1], idx1_hbm, gouts[2 * nj - 2], 128),
                    (aux1_hbm, idx1_hbm, gouts[2 * nj - 1], 128)]
            if with_emb:
                jobs.insert(0, (tbl_hbm.at[0], idx1_hbm, gouts[nj], 128))
            run_jobs(jobs)

    return k(tbl_stack, gidx, sidx, vals, aux0, aux1, idx0, idx1)


def _side_pre_body(q_ref, k_ref, v_ref, watt_ref, m2a_ref, m2b_ref,
                   wmean_ref, out_ref):
    w = watt_ref[...]
    m = jnp.dot(w, w.T, preferred_element_type=jnp.float32)
    q = jnp.dot(q_ref[...], m, preferred_element_type=jnp.float32)
    logits = jax.lax.dot_general(
        q.astype(jnp.bfloat16), k_ref[...].astype(jnp.bfloat16),
        (((1,), (1,)), ((), ())), preferred_element_type=jnp.float32)
    mx = jnp.max(logits, axis=1, keepdims=True)
    p = jnp.exp(logits - mx)
    sm = jnp.sum(p, axis=1, keepdims=True)
    att = jnp.dot(p.astype(jnp.bfloat16), v_ref[...].astype(jnp.bfloat16),
                  preferred_element_type=jnp.float32) / sm
    h1 = jnp.dot(att, w, preferred_element_type=jnp.float32)
    h2 = jnp.dot(m2a_ref[...] * m2b_ref[...], wmean_ref[...],
                 preferred_element_type=jnp.float32)
    out_ref[...] = jnp.concatenate([h1, h2], axis=1)


def _side_pre(q, k, v, watt, m2a, m2b, wmean):
    bsz = q.shape[0]
    bq = 256
    qmap = lambda i: (i, 0)
    full = lambda i: (0, 0)
    return pl.pallas_call(
        _side_pre_body,
        grid=(bsz // bq,),
        in_specs=[
            pl.BlockSpec((bq, DD), qmap),
            pl.BlockSpec((N_U, DD), full),
            pl.BlockSpec((N_U, DD), full),
            pl.BlockSpec((DD, DD), full),
            pl.BlockSpec((bq, DD), qmap),
            pl.BlockSpec((bq, DD), qmap),
            pl.BlockSpec((DD, DD), full),
        ],
        out_specs=pl.BlockSpec((bq, 2 * DD), qmap),
        out_shape=jax.ShapeDtypeStruct((bsz, 2 * DD), jnp.float32),
    )(q, k, v, watt, m2a, m2b, wmean)


def _final_body(hs_ref, oa_ref, ob_ref, wobs_ref, out_ref):
    ho = jnp.tanh(jnp.dot(oa_ref[...] * ob_ref[...], wobs_ref[...],
                          preferred_element_type=jnp.float32))
    h = jnp.tanh(jnp.concatenate([hs_ref[...], ho], axis=1))
    n = jnp.sqrt(jnp.sum(h * h, axis=1, keepdims=True))
    out_ref[...] = h / jnp.maximum(n, 1e-12)


def _final(hs, oa, ob, wobs):
    bsz = hs.shape[0]
    bq = 256
    qmap = lambda i: (i, 0)
    full = lambda i: (0, 0)
    return pl.pallas_call(
        _final_body,
        grid=(bsz // bq,),
        in_specs=[
            pl.BlockSpec((bq, 2 * DD), qmap),
            pl.BlockSpec((bq, DD), qmap),
            pl.BlockSpec((bq, DD), qmap),
            pl.BlockSpec((DD, DD), full),
        ],
        out_specs=pl.BlockSpec((bq, 3 * DD), qmap),
        out_shape=jax.ShapeDtypeStruct((bsz, 3 * DD), jnp.float32),
    )(hs, oa, ob, wobs)


def kernel(user_emb, item_emb, W_att_user, W_att_item, W_mean_user,
           W_mean_item, W_obs_user, W_obs_item, sample_user_n_j,
           sample_item_n_j, obs_user_n_j, obs_item_n_j, adj_values,
           obs_adj_values, users, pos_items, neg_items, obs_users,
           obs_pos_items, obs_neg_items, adj_indices, obs_adj_indices):
    tbl_stack = jnp.stack([item_emb, user_emb])
    rs = lambda x: x.reshape(_SC_NS, _NCH, _CK)
    gidx1 = jnp.stack([rs(adj_indices[1]), rs(adj_indices[0])])
    sidx1 = jnp.stack([rs(adj_indices[0]), rs(adj_indices[1])])
    gidx2 = jnp.stack([rs(obs_adj_indices[1]), rs(obs_adj_indices[0])])
    sidx2 = jnp.stack([rs(obs_adj_indices[0]), rs(obs_adj_indices[1])])
    idx2 = jnp.concatenate([pos_items, neg_items])
    idxo2 = jnp.concatenate([obs_pos_items, obs_neg_items])

    spmm_o, qu, m2a_u, m2b_u, qi, m2a_i, m2b_i = _sc_phase(
        tbl_stack, gidx1, sidx1, adj_values,
        sample_user_n_j, sample_item_n_j, users, idx2, True)
    _, oa_u, ob_u, oa_i, ob_i = _sc_phase(
        tbl_stack, gidx2, sidx2, obs_adj_values,
        obs_user_n_j, obs_item_n_j, obs_users, idxo2, False)

    hs_u = _side_pre(qu, spmm_o[1], item_emb, W_att_user,
                     m2a_u, m2b_u, W_mean_user)
    hs_pn = _side_pre(qi, spmm_o[0], user_emb, W_att_item,
                      m2a_i, m2b_i, W_mean_item)
    h_u = _final(hs_u, oa_u, ob_u, W_obs_user)
    h_pn = _final(hs_pn, oa_i, ob_i, W_obs_item)
    return h_u, h_pn[:1024], h_pn[1024:]
